# baseline (speedup 1.0000x reference)
import jax
import jax.numpy as jnp
from jax.experimental import pallas as pl

N_HIDDEN = 256
N_MPC = 10
N_CLASS = 100
N_MOTIF = N_MPC * N_CLASS
N_MOTIF_PAD = 1024
TEMP = 0.2
EPS = 1e-4

BLK = 512


def _motif_loss_kernel(z_ref, ylo_ref, mt_ref, out_ref):
    i = pl.program_id(0)
    z = z_ref[...]
    mt = mt_ref[...]
    ylo = ylo_ref[...]

    zsq = jnp.sum(z * z, axis=1, keepdims=True)
    msq = jnp.sum(mt * mt, axis=0, keepdims=True)
    xp = jnp.dot(z.astype(jnp.bfloat16), mt.astype(jnp.bfloat16),
                 preferred_element_type=jnp.float32)
    d = zsq + msq - 2.0 * xp

    r = (d + 1.0) / (d + EPS)
    r2 = r * r
    s = r2 * r2 * r

    col = jax.lax.broadcasted_iota(
        jnp.int32, (BLK, N_MOTIF_PAD), 1).astype(jnp.float32)
    valid = col < float(N_MOTIF)
    pos = (col >= ylo) & (col < ylo + float(N_MPC))

    s = jnp.where(valid, s, 0.0)
    s_pos = jnp.where(pos, s, 0.0)
    total = jnp.sum(s, axis=1, keepdims=True)
    pos_sum = jnp.sum(s_pos, axis=1, keepdims=True)
    pos_max = jnp.max(jnp.where(pos, s, -jnp.inf), axis=1, keepdims=True)

    terms = jnp.log(pos_max) - jnp.log(total - pos_sum + pos_max)
    acc = jnp.sum(terms, keepdims=True).reshape(1, 1)

    @pl.when(i == 0)
    def _init():
        out_ref[...] = jnp.zeros((1, 1), jnp.float32)

    out_ref[...] += acc


@jax.jit
def kernel(z, y, Motif_Vector):
    b = z.shape[0]
    mt = jnp.pad(Motif_Vector.T, ((0, 0), (0, N_MOTIF_PAD - N_MOTIF)))
    ylo = (y.astype(jnp.float32) * float(N_MPC)).reshape(b, 1)

    grid = b // BLK
    total = pl.pallas_call(
        _motif_loss_kernel,
        grid=(grid,),
        in_specs=[
            pl.BlockSpec((BLK, N_HIDDEN), lambda i: (i, 0)),
            pl.BlockSpec((BLK, 1), lambda i: (i, 0)),
            pl.BlockSpec((N_HIDDEN, N_MOTIF_PAD), lambda i: (0, 0)),
        ],
        out_specs=pl.BlockSpec((1, 1), lambda i: (0, 0)),
        out_shape=jax.ShapeDtypeStruct((1, 1), jnp.float32),
    )(z, ylo, mt)

    return -total[0, 0] / b


# speedup vs baseline: 1.0220x; 1.0220x over previous
import jax
import jax.numpy as jnp
from jax.experimental import pallas as pl
from jax.experimental.pallas import tpu as pltpu

N_HIDDEN = 256
N_MPC = 10
N_CLASS = 100
N_MOTIF = N_MPC * N_CLASS
N_MOTIF_PAD = 1024
N_PAD_COLS = N_MOTIF_PAD - N_MOTIF
TEMP = 0.2
EPS = 1e-4
PAD_VAL = 512.0

BLK = 512


def _motif_loss_kernel(z_ref, ylo_ref, mt_ref, out_ref):
    i = pl.program_id(0)
    z = z_ref[...]
    mt = mt_ref[...]
    ylo = ylo_ref[...]

    zsq = jnp.sum(z * z, axis=1, keepdims=True)
    msq = jnp.sum(mt * mt, axis=0, keepdims=True)
    xp = jnp.dot(z.astype(jnp.bfloat16), mt.astype(jnp.bfloat16),
                 preferred_element_type=jnp.float32)
    d = (zsq + msq) - 2.0 * xp

    r = (d + 1.0) / (d + EPS)
    r2 = r * r
    s = r2 * r2 * r

    col = jax.lax.broadcasted_iota(jnp.int32, (BLK, N_MOTIF_PAD), 1)
    pos = (col >= ylo) & (col < ylo + N_MPC)

    neg = jnp.sum(jnp.where(pos, 0.0, s), axis=1, keepdims=True) - float(N_PAD_COLS)
    pos_max = jnp.max(jnp.where(pos, s, -jnp.inf), axis=1, keepdims=True)

    terms = jnp.log(pos_max) - jnp.log(neg + pos_max)
    acc = jnp.sum(terms, keepdims=True).reshape(1, 1)

    @pl.when(i == 0)
    def _init():
        out_ref[...] = jnp.zeros((1, 1), jnp.float32)

    out_ref[...] += acc


@jax.jit
def kernel(z, y, Motif_Vector):
    b = z.shape[0]
    mt = jnp.pad(Motif_Vector.T, ((0, 0), (0, N_PAD_COLS)),
                 constant_values=PAD_VAL)
    ylo = (y.astype(jnp.int32) * N_MPC).reshape(b, 1)

    grid = b // BLK
    total = pl.pallas_call(
        _motif_loss_kernel,
        grid=(grid,),
        in_specs=[
            pl.BlockSpec((BLK, N_HIDDEN), lambda i: (i, 0)),
            pl.BlockSpec((BLK, 1), lambda i: (i, 0)),
            pl.BlockSpec((N_HIDDEN, N_MOTIF_PAD), lambda i: (0, 0)),
        ],
        out_specs=pl.BlockSpec((1, 1), lambda i: (0, 0)),
        out_shape=jax.ShapeDtypeStruct((1, 1), jnp.float32),
    )(z, ylo, mt)

    return -total[0, 0] / b
